# baseline (speedup 1.0000x reference)
import functools

import jax
import jax.numpy as jnp
from jax import lax
from jax.experimental import pallas as pl
from jax.experimental.pallas import tpu as pltpu
from jax.experimental.pallas import tpu_sc as plsc

NA = 8192
NB = 8192
D = 128
E = 131072
ALPHA = 0.2

NC = 2
NS = 16
NW = NC * NS
EPS = E // NS
ROWS_PW = NA // NW
ZCH = 32768
FILL_PW = ROWS_PW * NB
CAP = EPS + 128
NCH = CAP // 128

_f32 = jnp.float32
_i32 = jnp.int32


def _tc1_body(a_ref, b_ref, da_ref, w_ref, atta_ref, dw_ref, attda_ref,
              u_ref, v_ref, alpha_ref, m_ref):
    hi = jax.lax.Precision.HIGHEST
    w = w_ref[...]
    atta = atta_ref[...]
    wu = jnp.dot(w, atta[:D, :], preferred_element_type=_f32, precision=hi)
    wv = jnp.dot(w, atta[D:, :], preferred_element_type=_f32, precision=hi)
    u = jnp.dot(a_ref[...], wu, preferred_element_type=_f32, precision=hi)
    v = jnp.dot(b_ref[...], wv, preferred_element_type=_f32, precision=hi)
    u_ref[...] = u
    v_ref[...] = v

    dw = dw_ref[...]
    attda = attda_ref[...]
    wd1 = jnp.dot(dw, attda[:D, :], preferred_element_type=_f32, precision=hi)
    wd2 = jnp.dot(dw, attda[D:, :], preferred_element_type=_f32, precision=hi)
    de = (jnp.dot(a_ref[...], wd1, preferred_element_type=_f32, precision=hi)
          + jnp.dot(da_ref[...], wd2, preferred_element_type=_f32, precision=hi))
    alpha_ref[...] = 1.0 / (1.0 + jnp.exp(-de))

    m = jnp.max(u) + jnp.max(v)
    m = jnp.where(m >= 0.0, m, ALPHA * m)
    m_ref[...] = jnp.full((1, 128), m, dtype=_f32)


_tc1 = pl.pallas_call(
    _tc1_body,
    out_shape=(
        jax.ShapeDtypeStruct((NA, 1), _f32),
        jax.ShapeDtypeStruct((NB, 1), _f32),
        jax.ShapeDtypeStruct((NA, 1), _f32),
        jax.ShapeDtypeStruct((1, 128), _f32),
    ),
)


_sc_mesh = plsc.VectorSubcoreMesh(core_axis_name="c", subcore_axis_name="s")
_sc_params = pltpu.CompilerParams(needs_layout_passes=False)


@functools.partial(
    pl.kernel,
    out_type=jax.ShapeDtypeStruct((NA * NB,), _f32),
    mesh=_sc_mesh,
    compiler_params=_sc_params,
    scratch_types=[
        pltpu.VMEM((ZCH,), _f32),
        pltpu.VMEM((NA,), _f32),
        pltpu.VMEM((NB,), _f32),
        pltpu.VMEM((16,), _f32),
        pltpu.VMEM((EPS,), _i32),
        pltpu.VMEM((EPS,), _i32),
        pltpu.VMEM((CAP,), _i32),
        pltpu.VMEM((CAP,), _f32),
        pltpu.VMEM((NCH, 128), _i32),
        pltpu.VMEM((NCH, 128), _f32),
        pltpu.SemaphoreType.DMA,
        pltpu.SemaphoreType.DMA,
    ],
)
def _sc_mega(src_h, dst_h, u_h, v_h, m_h, pmat_h,
             zbuf, u_v, v_v, m_v, src_v, dst_v, flat_l, p_l, idx2d, p2d,
             fsem, ssem):
    c = lax.axis_index("c")
    s = lax.axis_index("s")
    wid = s * NC + c
    zero16 = jnp.zeros((16,), _f32)

    def _zb(i, carry):
        zbuf[pl.ds(i * 16, 16)] = zero16
        return carry

    lax.fori_loop(0, ZCH // 16, _zb, 0)
    fill_base = wid * FILL_PW
    fills = []
    for k in range(FILL_PW // ZCH):
        fills.append(
            pltpu.async_copy(zbuf, pmat_h.at[pl.ds(fill_base + k * ZCH, ZCH)],
                             fsem)
        )

    ebase = s * EPS
    pltpu.sync_copy(u_h, u_v)
    pltpu.sync_copy(v_h, v_v)
    pltpu.sync_copy(m_h.at[pl.ds(0, 16)], m_v)
    pltpu.sync_copy(src_h.at[pl.ds(ebase, EPS)], src_v)
    pltpu.sync_copy(dst_h.at[pl.ds(ebase, EPS)], dst_v)

    m16 = m_v[...]
    half = c * (NA // NC)

    def _grp(g, off):
        srcs = src_v[pl.ds(g * 16, 16)]
        dsts = dst_v[pl.ds(g * 16, 16)]
        us = plsc.load_gather(u_v, [srcs])
        vs = plsc.load_gather(v_v, [dsts])
        x = us + vs
        e = jnp.where(x >= 0.0, x, x * ALPHA)
        p = jnp.exp(e - m16)
        flat = srcs * NB + dsts
        mine = (srcs >> 12) == c
        plsc.store_compressed(flat_l.at[pl.ds(off, 16)], flat, mask=mine)
        plsc.store_compressed(p_l.at[pl.ds(off, 16)], p, mask=mine)
        cnt = jnp.max(plsc.all_reduce_population_count(mine))
        return off + cnt

    n = lax.fori_loop(0, EPS // 16, _grp, jnp.int32(0))

    nch = n // 128 + 1

    @pl.when(n > 0)
    def _scatter_phase():
        lastf = jnp.full((16,), flat_l[pl.ds(n - 1, 16)][0], dtype=_i32)
        lastp = jnp.full((16,), p_l[pl.ds(n - 1, 16)][0], dtype=_f32)
        for k in range(8):
            flat_l[pl.ds(n + k * 16, 16)] = lastf
            p_l[pl.ds(n + k * 16, 16)] = lastp

        def _rp(j, carry):
            for k in range(8):
                o = j * 128 + k * 16
                idx2d[j, pl.ds(k * 16, 16)] = flat_l[pl.ds(o, 16)]
                p2d[j, pl.ds(k * 16, 16)] = p_l[pl.ds(o, 16)]
            return carry

        lax.fori_loop(0, nch, _rp, 0)

    for f in fills:
        f.wait()
    plsc.subcore_barrier()

    @pl.when(n > 0)
    def _scatter_fire():
        def _fire(j, carry):
            pltpu.async_copy(p2d.at[j], pmat_h.at[idx2d.at[j]], ssem)
            return carry

        lax.fori_loop(0, nch, _fire, 0)

        def _drain(j, carry):
            pltpu.make_async_copy(p2d.at[j], pmat_h.at[idx2d.at[j]], ssem).wait()
            return carry

        lax.fori_loop(0, nch, _drain, 0)


_BR = 256


def _tc2_body(pm_ref, bdis_ref, alpha_ref, dadis_ref, attn_ref, out_ref):
    blk = pm_ref[...]
    ssum = jnp.sum(blk, axis=1, keepdims=True)
    empty = ssum <= 0.0
    rinv = 1.0 / jnp.where(empty, 1.0, ssum)
    attn = jnp.where(empty, _f32(1.0 / NB), blk * rinv)
    attn_ref[...] = attn
    acc = jnp.dot(attn, bdis_ref[...], preferred_element_type=_f32)
    a = alpha_ref[...]
    x = (1.0 - a) * acc + a * dadis_ref[...]
    y = jnp.tanh(x)
    n = jnp.maximum(jnp.sum(jnp.abs(y), axis=1, keepdims=True), 1e-12)
    out_ref[...] = y / n


_tc2 = pl.pallas_call(
    _tc2_body,
    grid=(NA // _BR,),
    in_specs=[
        pl.BlockSpec((_BR, NB), lambda i: (i, 0)),
        pl.BlockSpec((NB, D), lambda i: (0, 0)),
        pl.BlockSpec((_BR, 1), lambda i: (i, 0)),
        pl.BlockSpec((_BR, D), lambda i: (i, 0)),
    ],
    out_specs=(
        pl.BlockSpec((_BR, NB), lambda i: (i, 0)),
        pl.BlockSpec((_BR, D), lambda i: (i, 0)),
    ),
    out_shape=(
        jax.ShapeDtypeStruct((NA, NB), _f32),
        jax.ShapeDtypeStruct((NA, D), _f32),
    ),
)


def kernel(a_feats, b_feats, da_feats, b_dis, da_dis, adj_ab, W, att_a, dW, att_da):
    src = adj_ab[0].astype(jnp.int32)
    dst = adj_ab[1].astype(jnp.int32)

    u, v, alpha, m = _tc1(a_feats, b_feats, da_feats, W, att_a, dW, att_da)
    u1 = u.reshape(NA)
    v1 = v.reshape(NB)
    m1 = m.reshape(128)

    pmat = _sc_mega(src, dst, u1, v1, m1).reshape(NA, NB)

    attention, new_a_dis = _tc2(pmat, b_dis, alpha, da_dis)
    return (new_a_dis, attention, alpha.reshape(NA))


# speedup vs baseline: 1.0620x; 1.0620x over previous
import functools

import jax
import jax.numpy as jnp
from jax import lax
from jax.experimental import pallas as pl
from jax.experimental.pallas import tpu as pltpu
from jax.experimental.pallas import tpu_sc as plsc

NA = 8192
NB = 8192
D = 128
E = 131072
ALPHA = 0.2

NC = 2
NS = 16
NW = NC * NS
EPW = E // NW
ROWS_PW = NA // NW
ZCH = 32768
FILL_PW = ROWS_PW * NB

_f32 = jnp.float32


def _tc1_body(a_ref, b_ref, da_ref, w_ref, atta_ref, dw_ref, attda_ref,
              u_ref, v_ref, alpha_ref, m_ref):
    w = w_ref[...]
    atta = atta_ref[...]
    wu = jnp.dot(w, atta[:D, :], preferred_element_type=_f32, precision=jax.lax.Precision.HIGHEST)
    wv = jnp.dot(w, atta[D:, :], preferred_element_type=_f32, precision=jax.lax.Precision.HIGHEST)
    u = jnp.dot(a_ref[...], wu, preferred_element_type=_f32, precision=jax.lax.Precision.HIGHEST)
    v = jnp.dot(b_ref[...], wv, preferred_element_type=_f32, precision=jax.lax.Precision.HIGHEST)
    u_ref[...] = u
    v_ref[...] = v

    dw = dw_ref[...]
    attda = attda_ref[...]
    wd1 = jnp.dot(dw, attda[:D, :], preferred_element_type=_f32, precision=jax.lax.Precision.HIGHEST)
    wd2 = jnp.dot(dw, attda[D:, :], preferred_element_type=_f32, precision=jax.lax.Precision.HIGHEST)
    de = (jnp.dot(a_ref[...], wd1, preferred_element_type=_f32, precision=jax.lax.Precision.HIGHEST)
          + jnp.dot(da_ref[...], wd2, preferred_element_type=_f32, precision=jax.lax.Precision.HIGHEST))
    alpha_ref[...] = 1.0 / (1.0 + jnp.exp(-de))

    m = jnp.max(u) + jnp.max(v)
    m = jnp.where(m >= 0.0, m, ALPHA * m)
    m_ref[...] = jnp.full((1, 128), m, dtype=_f32)


_tc1 = pl.pallas_call(
    _tc1_body,
    out_shape=(
        jax.ShapeDtypeStruct((NA, 1), _f32),
        jax.ShapeDtypeStruct((NB, 1), _f32),
        jax.ShapeDtypeStruct((NA, 1), _f32),
        jax.ShapeDtypeStruct((1, 128), _f32),
    ),
)


_sc_mesh = plsc.VectorSubcoreMesh(core_axis_name="c", subcore_axis_name="s")
_sc_params = pltpu.CompilerParams(needs_layout_passes=False)


@functools.partial(
    pl.kernel,
    out_type=(),
    mesh=_sc_mesh,
    compiler_params=_sc_params,
    scratch_types=[
        pltpu.VMEM((ZCH,), _f32),
        pltpu.SemaphoreType.DMA,
    ],
)
def _sc_fill(attn_ref, zbuf, sem):
    c = lax.axis_index("c")
    s = lax.axis_index("s")
    wid = s * NC + c
    zero16 = jnp.zeros((16,), _f32)

    def _zb(i, carry):
        zbuf[pl.ds(i * 16, 16)] = zero16
        return carry

    lax.fori_loop(0, ZCH // 16, _zb, 0)
    base = wid * FILL_PW
    descs = []
    for k in range(FILL_PW // ZCH):
        descs.append(
            pltpu.async_copy(zbuf, attn_ref.at[pl.ds(base + k * ZCH, ZCH)], sem)
        )
    for d in descs:
        d.wait()


_GRP = EPW // 128


@functools.partial(
    pl.kernel,
    out_type=(),
    mesh=_sc_mesh,
    compiler_params=_sc_params,
    scratch_types=[
        pltpu.VMEM((NA,), _f32),
        pltpu.VMEM((NB,), _f32),
        pltpu.VMEM((16,), _f32),
        pltpu.VMEM((EPW,), jnp.int32),
        pltpu.VMEM((EPW,), jnp.int32),
        pltpu.VMEM((_GRP, 128), _f32),
        pltpu.VMEM((_GRP, 128), jnp.int32),
        pltpu.SemaphoreType.DMA,
    ],
)
def _sc_scatter(src_h, dst_h, u_h, v_h, m_h, attn_ref,
                u_v, v_v, m_v, src_v, dst_v, p2d, idx2d, sem):
    c = lax.axis_index("c")
    s = lax.axis_index("s")
    wid = s * NC + c
    ebase = wid * EPW

    pltpu.sync_copy(u_h, u_v)
    pltpu.sync_copy(v_h, v_v)
    pltpu.sync_copy(m_h.at[pl.ds(0, 16)], m_v)
    pltpu.sync_copy(src_h.at[pl.ds(ebase, EPW)], src_v)
    pltpu.sync_copy(dst_h.at[pl.ds(ebase, EPW)], dst_v)

    m16 = m_v[...]

    def _grp(j, carry):
        for k in range(8):
            off = j * 128 + k * 16
            srcs = src_v[pl.ds(off, 16)]
            dsts = dst_v[pl.ds(off, 16)]
            us = plsc.load_gather(u_v, [srcs])
            vs = plsc.load_gather(v_v, [dsts])
            x = us + vs
            e = jnp.where(x >= 0.0, x, x * ALPHA)
            p = jnp.exp(e - m16)
            p2d[j, pl.ds(k * 16, 16)] = p
            idx2d[j, pl.ds(k * 16, 16)] = srcs * NB + dsts
        return carry

    lax.fori_loop(0, _GRP, _grp, 0)

    descs = []
    for j in range(_GRP):
        descs.append(
            pltpu.async_copy(p2d.at[j], attn_ref.at[idx2d.at[j]], sem)
        )
    for d in descs:
        d.wait()


_BR = 256


def _tc2_body(pm_ref, bdis_ref, alpha_ref, dadis_ref, attn_ref, out_ref):
    blk = pm_ref[...]
    ssum = jnp.sum(blk, axis=1, keepdims=True)
    empty = ssum <= 0.0
    rinv = 1.0 / jnp.where(empty, 1.0, ssum)
    attn = jnp.where(empty, _f32(1.0 / NB), blk * rinv)
    attn_ref[...] = attn
    acc = jnp.dot(attn, bdis_ref[...], preferred_element_type=_f32)
    a = alpha_ref[...]
    x = (1.0 - a) * acc + a * dadis_ref[...]
    y = jnp.tanh(x)
    n = jnp.maximum(jnp.sum(jnp.abs(y), axis=1, keepdims=True), 1e-12)
    out_ref[...] = y / n


_tc2 = pl.pallas_call(
    _tc2_body,
    grid=(NA // _BR,),
    in_specs=[
        pl.BlockSpec((_BR, NB), lambda i: (i, 0)),
        pl.BlockSpec((NB, D), lambda i: (0, 0)),
        pl.BlockSpec((_BR, 1), lambda i: (i, 0)),
        pl.BlockSpec((_BR, D), lambda i: (i, 0)),
    ],
    out_specs=(
        pl.BlockSpec((_BR, NB), lambda i: (i, 0)),
        pl.BlockSpec((_BR, D), lambda i: (i, 0)),
    ),
    out_shape=(
        jax.ShapeDtypeStruct((NA, NB), _f32),
        jax.ShapeDtypeStruct((NA, D), _f32),
    ),
)


def kernel(a_feats, b_feats, da_feats, b_dis, da_dis, adj_ab, W, att_a, dW, att_da):
    src = adj_ab[0].astype(jnp.int32)
    dst = adj_ab[1].astype(jnp.int32)

    u, v, alpha, m = _tc1(a_feats, b_feats, da_feats, W, att_a, dW, att_da)
    u1 = u.reshape(NA)
    v1 = v.reshape(NB)
    m1 = m.reshape(128)

    attn_ref = jax.new_ref(jax.lax.empty((NA * NB,), _f32))
    _sc_fill(attn_ref)
    _sc_scatter(src, dst, u1, v1, m1, attn_ref)
    pmat = jax.freeze(attn_ref).reshape(NA, NB)

    attention, new_a_dis = _tc2(pmat, b_dis, alpha, da_dis)
    return (new_a_dis, attention, alpha.reshape(NA))


# speedup vs baseline: 1.4953x; 1.4080x over previous
import functools

import jax
import jax.numpy as jnp
from jax import lax
from jax.experimental import pallas as pl
from jax.experimental.pallas import tpu as pltpu
from jax.experimental.pallas import tpu_sc as plsc

NA = 8192
NB = 8192
D = 128
E = 131072
ALPHA = 0.2

NC = 2
NS = 16
NW = NC * NS
EPS = E // NS
ROWS_PW = NA // NW
ZCH = 32768
FILL_PW = ROWS_PW * NB
CAP = EPS + 128
NCH = CAP // 128

_f32 = jnp.float32
_i32 = jnp.int32


def _tc1_body(a_ref, b_ref, da_ref, w_ref, atta_ref, dw_ref, attda_ref,
              u_ref, v_ref, alpha_ref, m_ref):
    hi = jax.lax.Precision.HIGHEST
    w = w_ref[...]
    atta = atta_ref[...]
    wu = jnp.dot(w, atta[:D, :], preferred_element_type=_f32, precision=hi)
    wv = jnp.dot(w, atta[D:, :], preferred_element_type=_f32, precision=hi)
    u = jnp.dot(a_ref[...], wu, preferred_element_type=_f32, precision=hi)
    v = jnp.dot(b_ref[...], wv, preferred_element_type=_f32, precision=hi)
    u_ref[...] = u
    v_ref[...] = v

    dw = dw_ref[...]
    attda = attda_ref[...]
    wd1 = jnp.dot(dw, attda[:D, :], preferred_element_type=_f32, precision=hi)
    wd2 = jnp.dot(dw, attda[D:, :], preferred_element_type=_f32, precision=hi)
    de = (jnp.dot(a_ref[...], wd1, preferred_element_type=_f32, precision=hi)
          + jnp.dot(da_ref[...], wd2, preferred_element_type=_f32, precision=hi))
    alpha_ref[...] = 1.0 / (1.0 + jnp.exp(-de))

    m = jnp.max(u) + jnp.max(v)
    m = jnp.where(m >= 0.0, m, ALPHA * m)
    m_ref[...] = jnp.full((1, 128), m, dtype=_f32)


_tc1 = pl.pallas_call(
    _tc1_body,
    out_shape=(
        jax.ShapeDtypeStruct((NA, 1), _f32),
        jax.ShapeDtypeStruct((NB, 1), _f32),
        jax.ShapeDtypeStruct((NA, 1), _f32),
        jax.ShapeDtypeStruct((1, 128), _f32),
    ),
)


_sc_mesh = plsc.VectorSubcoreMesh(core_axis_name="c", subcore_axis_name="s")
_sc_params = pltpu.CompilerParams(needs_layout_passes=False)


@functools.partial(
    pl.kernel,
    out_type=jax.ShapeDtypeStruct((NA * NB,), _f32),
    mesh=_sc_mesh,
    compiler_params=_sc_params,
    scratch_types=[
        pltpu.VMEM((ZCH,), _f32),
        pltpu.VMEM((NA,), _f32),
        pltpu.VMEM((NB,), _f32),
        pltpu.VMEM((16,), _f32),
        pltpu.VMEM((EPS,), _i32),
        pltpu.VMEM((EPS,), _i32),
        pltpu.VMEM((CAP,), _i32),
        pltpu.VMEM((CAP,), _f32),
        pltpu.VMEM((NCH, 128), _i32),
        pltpu.VMEM((NCH, 128), _f32),
        pltpu.SemaphoreType.DMA,
        pltpu.SemaphoreType.DMA,
    ],
)
def _sc_mega(src_h, dst_h, u_h, v_h, m_h, pmat_h,
             zbuf, u_v, v_v, m_v, src_v, dst_v, flat_l, p_l, idx2d, p2d,
             fsem, ssem):
    c = lax.axis_index("c")
    s = lax.axis_index("s")
    wid = s * NC + c
    zero16 = jnp.zeros((16,), _f32)

    def _zb(i, carry):
        zbuf[pl.ds(i * 16, 16)] = zero16
        return carry

    lax.fori_loop(0, ZCH // 16, _zb, 0)
    fill_base = wid * FILL_PW
    fills = []
    for k in range(FILL_PW // ZCH):
        fills.append(
            pltpu.async_copy(zbuf, pmat_h.at[pl.ds(fill_base + k * ZCH, ZCH)],
                             fsem)
        )

    ebase = s * EPS
    pltpu.sync_copy(u_h, u_v)
    pltpu.sync_copy(v_h, v_v)
    pltpu.sync_copy(m_h.at[pl.ds(0, 16)], m_v)
    pltpu.sync_copy(src_h.at[pl.ds(ebase, EPS)], src_v)
    pltpu.sync_copy(dst_h.at[pl.ds(ebase, EPS)], dst_v)

    m16 = m_v[...]
    half = c * (NA // NC)

    def _grp(g, off):
        srcs = src_v[pl.ds(g * 16, 16)]
        dsts = dst_v[pl.ds(g * 16, 16)]
        us = plsc.load_gather(u_v, [srcs])
        vs = plsc.load_gather(v_v, [dsts])
        x = us + vs
        e = jnp.where(x >= 0.0, x, x * ALPHA)
        p = jnp.exp(e - m16)
        flat = srcs * NB + dsts
        mine = (srcs >> 12) == c
        plsc.store_compressed(flat_l.at[pl.ds(off, 16)], flat, mask=mine)
        plsc.store_compressed(p_l.at[pl.ds(off, 16)], p, mask=mine)
        cnt = jnp.max(plsc.all_reduce_population_count(mine))
        return off + cnt

    n = lax.fori_loop(0, EPS // 16, _grp, jnp.int32(0))

    nch = n // 128 + 1

    @pl.when(n > 0)
    def _scatter_phase():
        lastf = jnp.full((16,), flat_l[pl.ds(n - 1, 16)][0], dtype=_i32)
        lastp = jnp.full((16,), p_l[pl.ds(n - 1, 16)][0], dtype=_f32)
        for k in range(8):
            flat_l[pl.ds(n + k * 16, 16)] = lastf
            p_l[pl.ds(n + k * 16, 16)] = lastp

        def _rp(j, carry):
            for k in range(8):
                o = j * 128 + k * 16
                idx2d[j, pl.ds(k * 16, 16)] = flat_l[pl.ds(o, 16)]
                p2d[j, pl.ds(k * 16, 16)] = p_l[pl.ds(o, 16)]
            return carry

        lax.fori_loop(0, nch, _rp, 0)

    for f in fills:
        f.wait()
    plsc.subcore_barrier()

    @pl.when(n > 0)
    def _scatter_fire():
        def _fire(j, carry):
            pltpu.async_copy(p2d.at[j], pmat_h.at[idx2d.at[j]], ssem)
            return carry

        lax.fori_loop(0, nch, _fire, 0)

        def _drain(j, carry):
            pltpu.make_async_copy(p2d.at[j], pmat_h.at[idx2d.at[j]], ssem).wait()
            return carry

        lax.fori_loop(0, nch, _drain, 0)


_BR = 256


def _tc2_body(pm_ref, bdis_ref, alpha_ref, dadis_ref, attn_ref, out_ref):
    blk = pm_ref[...].reshape(_BR, NB)
    ssum = jnp.sum(blk, axis=1, keepdims=True)
    empty = ssum <= 0.0
    rinv = 1.0 / jnp.where(empty, 1.0, ssum)
    attn = jnp.where(empty, _f32(1.0 / NB), blk * rinv)
    attn_ref[...] = attn
    acc = jnp.dot(attn, bdis_ref[...], preferred_element_type=_f32)
    a = alpha_ref[...]
    x = (1.0 - a) * acc + a * dadis_ref[...]
    y = jnp.tanh(x)
    n = jnp.maximum(jnp.sum(jnp.abs(y), axis=1, keepdims=True), 1e-12)
    out_ref[...] = y / n


_tc2 = pl.pallas_call(
    _tc2_body,
    grid=(NA // _BR,),
    in_specs=[
        pl.BlockSpec((_BR * NB,), lambda i: (i,)),
        pl.BlockSpec((NB, D), lambda i: (0, 0)),
        pl.BlockSpec((_BR, 1), lambda i: (i, 0)),
        pl.BlockSpec((_BR, D), lambda i: (i, 0)),
    ],
    out_specs=(
        pl.BlockSpec((_BR, NB), lambda i: (i, 0)),
        pl.BlockSpec((_BR, D), lambda i: (i, 0)),
    ),
    out_shape=(
        jax.ShapeDtypeStruct((NA, NB), _f32),
        jax.ShapeDtypeStruct((NA, D), _f32),
    ),
)


def kernel(a_feats, b_feats, da_feats, b_dis, da_dis, adj_ab, W, att_a, dW, att_da):
    src = adj_ab[0].astype(jnp.int32)
    dst = adj_ab[1].astype(jnp.int32)

    u, v, alpha, m = _tc1(a_feats, b_feats, da_feats, W, att_a, dW, att_da)
    u1 = u.reshape(NA)
    v1 = v.reshape(NB)
    m1 = m.reshape(128)

    pmat = _sc_mega(src, dst, u1, v1, m1)

    attention, new_a_dis = _tc2(pmat, b_dis, alpha, da_dis)
    return (new_a_dis, attention, alpha.reshape(NA))
